# Initial kernel scaffold; baseline (speedup 1.0000x reference)
import jax
import jax.numpy as jnp
from jax.experimental import pallas as pl


def kernel(x, edge_index, edge_attr, batch, edge_emb1, edge_emb2, W1, b1, W2, b2, fc_W, fc_b):
    raise NotImplementedError("write your pallas kernel here")


# speedup vs baseline: 2.9783x; 2.9783x over previous
import functools

import jax
import jax.numpy as jnp
from jax import lax
from jax.experimental import pallas as pl
from jax.experimental.pallas import tpu as pltpu
from jax.experimental.pallas import tpu_sc as plsc

N, E, D, H, G = 10000, 320000, 128, 128, 64
NC, NS = 2, 16
NW = NC * NS
K = 128
CPT = 79
EPAD = NW * K * CPT
NP = 10240
RPT = NP // NS
RB = 1000
NBLK = N // RB

_mesh = plsc.VectorSubcoreMesh(core_axis_name="c", subcore_axis_name="s")


@functools.partial(
    pl.kernel,
    out_type=jax.ShapeDtypeStruct((NC, NP, D), jnp.float32),
    mesh=_mesh,
    scratch_types=[
        pltpu.VMEM((CPT, K), jnp.int32),
        pltpu.VMEM((CPT, K), jnp.int32),
        pltpu.VMEM((K, D), jnp.float32),
        pltpu.VMEM_SHARED((NP, D), jnp.float32),
        pltpu.SemaphoreType.DMA,
    ],
)
def _scatter_pass(table, srcw, dstw, zrows, out, src_v, dst_v, rows_v, acc, sem):
    c = lax.axis_index("c")
    s = lax.axis_index("s")
    wid = s * NC + c
    pltpu.sync_copy(zrows, acc.at[pl.ds(s * RPT, RPT)])
    pltpu.sync_copy(srcw.at[wid], src_v)
    pltpu.sync_copy(dstw.at[wid], dst_v)
    plsc.subcore_barrier()

    def body(j, carry):
        pltpu.async_copy(table.at[src_v.at[j]], rows_v, sem).wait()
        pltpu.sync_copy(rows_v, acc.at[dst_v.at[j]], add=True)
        return carry

    lax.fori_loop(0, CPT, body, 0)
    plsc.subcore_barrier()
    pltpu.sync_copy(acc.at[pl.ds(s * RPT, RPT)], out.at[c, pl.ds(s * RPT, RPT)])


def _mlp_body(a_ref, p_ref, h_ref, w1_ref, b1_ref, w2_ref, b2_ref, o_ref):
    a = a_ref[0] + a_ref[1] + p_ref[0] + p_ref[1] + h_ref[...]
    hid = jnp.dot(a, w1_ref[...], preferred_element_type=jnp.float32) + b1_ref[...]
    hid = jnp.maximum(hid, 0.0)
    o = jnp.dot(hid, w2_ref[...], preferred_element_type=jnp.float32) + b2_ref[...]
    o_ref[...] = jnp.maximum(o, 0.0)


def _mlp(a2, p2, hprev, W1, b1p, W2, b2r):
    return pl.pallas_call(
        _mlp_body,
        grid=(NBLK,),
        in_specs=[
            pl.BlockSpec((NC, RB, D), lambda i: (0, i, 0)),
            pl.BlockSpec((NC, RB, D), lambda i: (0, i, 0)),
            pl.BlockSpec((RB, D), lambda i: (i, 0)),
            pl.BlockSpec((D, 2 * D), lambda i: (0, 0)),
            pl.BlockSpec((1, 2 * D), lambda i: (0, 0)),
            pl.BlockSpec((2 * D, H), lambda i: (0, 0)),
            pl.BlockSpec((1, H), lambda i: (0, 0)),
        ],
        out_specs=pl.BlockSpec((RB, H), lambda i: (i, 0)),
        out_shape=jax.ShapeDtypeStruct((N, H), jnp.float32),
    )(a2, p2, hprev, W1, b1p, W2, b2r)


def _pool_body(h_ref, b_ref, fcw_ref, fcb_ref, o_ref, acc_ref):
    i = pl.program_id(0)

    @pl.when(i == 0)
    def _init():
        acc_ref[...] = jnp.zeros_like(acc_ref)

    b = jnp.reshape(b_ref[...], (1, RB))
    ids = lax.broadcasted_iota(jnp.int32, (G, RB), 0)
    onehot = (ids == b).astype(jnp.float32)
    acc_ref[...] += jnp.dot(onehot, h_ref[...], preferred_element_type=jnp.float32)

    @pl.when(i == NBLK - 1)
    def _fin():
        o_ref[...] = (
            jnp.dot(acc_ref[...], fcw_ref[...], preferred_element_type=jnp.float32)
            + fcb_ref[...]
        )


def _pool(h2, batchw, fc_W, fcbr):
    return pl.pallas_call(
        _pool_body,
        grid=(NBLK,),
        in_specs=[
            pl.BlockSpec((RB, D), lambda i: (i, 0)),
            pl.BlockSpec((1, 1, RB), lambda i: (i, 0, 0)),
            pl.BlockSpec((H, H), lambda i: (0, 0)),
            pl.BlockSpec((1, H), lambda i: (0, 0)),
        ],
        out_specs=pl.BlockSpec((G, H), lambda i: (0, 0)),
        out_shape=jax.ShapeDtypeStruct((G, H), jnp.float32),
        scratch_shapes=[pltpu.VMEM((G, H), jnp.float32)],
    )(h2, batchw, fc_W, fcbr)


def kernel(x, edge_index, edge_attr, batch, edge_emb1, edge_emb2, W1, b1, W2, b2, fc_W, fc_b):
    f32 = jnp.float32
    src = edge_index[0]
    dst = edge_index[1]
    code = edge_attr[:, 0] * 3 + edge_attr[:, 1]

    pad = EPAD - E
    srcw = jnp.concatenate([src, jnp.full((pad,), N, jnp.int32)]).reshape(NW, CPT, K)
    dstw = jnp.concatenate([dst, jnp.zeros((pad,), jnp.int32)]).reshape(NW, CPT, K)
    codew = jnp.concatenate([code, jnp.full((pad,), 15, jnp.int32)]).reshape(NW, CPT, K)

    idx = jnp.arange(16)
    T = jnp.where(
        (idx < 9)[:, None],
        edge_emb1[jnp.minimum(idx // 3, 5)] + edge_emb2[idx % 3],
        0.0,
    ).astype(f32)

    zrows = jnp.zeros((RPT, D), f32)
    tzero = jnp.zeros((8, D), f32)

    b1p = (b1 + (edge_emb1[0] + edge_emb2[0]) @ W1).reshape(1, 2 * D)
    b2r = b2.reshape(1, H)
    fcbr = fc_b.reshape(1, H)

    P2 = _scatter_pass(T, codew, dstw, zrows)
    A2 = _scatter_pass(jnp.concatenate([x, tzero], axis=0), srcw, dstw, zrows)
    h1 = _mlp(A2, P2, x, W1, b1p, W2, b2r)
    B2 = _scatter_pass(jnp.concatenate([h1, tzero], axis=0), srcw, dstw, zrows)
    h2 = _mlp(B2, P2, h1, W1, b1p, W2, b2r)

    batchw = batch.reshape(NBLK, 1, RB)
    return _pool(h2, batchw, fc_W, fcbr)


# speedup vs baseline: 6.5522x; 2.2000x over previous
import functools

import jax
import jax.numpy as jnp
from jax import lax
from jax.experimental import pallas as pl
from jax.experimental.pallas import tpu as pltpu
from jax.experimental.pallas import tpu_sc as plsc

N, E, D, H, G = 10000, 320000, 128, 128, 64
NC, NS = 2, 16
NW = NC * NS
K = 128
CPT = 79
EPAD = NW * K * CPT
NP = 10240
RPT = NP // NS
RB = 1000
NBLK = N // RB

_mesh = plsc.VectorSubcoreMesh(core_axis_name="c", subcore_axis_name="s")


@functools.partial(
    pl.kernel,
    out_type=jax.ShapeDtypeStruct((NC, NP, D), jnp.float32),
    mesh=_mesh,
    scratch_types=[
        pltpu.VMEM((CPT, K), jnp.int32),
        pltpu.VMEM((CPT, K), jnp.int32),
        pltpu.VMEM((K, D), jnp.float32),
        pltpu.VMEM_SHARED((NP, D), jnp.float32),
        pltpu.SemaphoreType.DMA,
    ],
)
def _scatter_pass(table, srcw, dstw, zrows, out, src_v, dst_v, rows_v, acc, sem):
    c = lax.axis_index("c")
    s = lax.axis_index("s")
    wid = s * NC + c
    pltpu.sync_copy(zrows, acc.at[pl.ds(s * RPT, RPT)])
    pltpu.sync_copy(srcw.at[wid], src_v)
    pltpu.sync_copy(dstw.at[wid], dst_v)
    plsc.subcore_barrier()

    def body(j, carry):
        pltpu.async_copy(table.at[src_v.at[j]], rows_v, sem).wait()
        pltpu.sync_copy(rows_v, acc.at[dst_v.at[j]], add=True)
        return carry

    lax.fori_loop(0, CPT, body, 0)
    plsc.subcore_barrier()
    pltpu.sync_copy(acc.at[pl.ds(s * RPT, RPT)], out.at[c, pl.ds(s * RPT, RPT)])


def _mlp_body(a_ref, p_ref, h_ref, w1_ref, b1_ref, w2_ref, b2_ref, o_ref):
    a = a_ref[0] + a_ref[1] + p_ref[0] + p_ref[1] + h_ref[...]
    hid = jnp.dot(a, w1_ref[...], preferred_element_type=jnp.float32) + b1_ref[...]
    hid = jnp.maximum(hid, 0.0)
    o = jnp.dot(hid, w2_ref[...], preferred_element_type=jnp.float32) + b2_ref[...]
    o_ref[...] = jnp.maximum(o, 0.0)


def _mlp(a2, p2, hprev, W1, b1p, W2, b2r):
    return pl.pallas_call(
        _mlp_body,
        grid=(NBLK,),
        in_specs=[
            pl.BlockSpec((NC, RB, D), lambda i: (0, i, 0)),
            pl.BlockSpec((NC, RB, D), lambda i: (0, i, 0)),
            pl.BlockSpec((RB, D), lambda i: (i, 0)),
            pl.BlockSpec((D, 2 * D), lambda i: (0, 0)),
            pl.BlockSpec((1, 2 * D), lambda i: (0, 0)),
            pl.BlockSpec((2 * D, H), lambda i: (0, 0)),
            pl.BlockSpec((1, H), lambda i: (0, 0)),
        ],
        out_specs=pl.BlockSpec((RB, H), lambda i: (i, 0)),
        out_shape=jax.ShapeDtypeStruct((N, H), jnp.float32),
    )(a2, p2, hprev, W1, b1p, W2, b2r)


def _pool_body(h_ref, b_ref, fcw_ref, fcb_ref, o_ref, acc_ref):
    i = pl.program_id(0)

    @pl.when(i == 0)
    def _init():
        acc_ref[...] = jnp.zeros_like(acc_ref)

    b = jnp.reshape(b_ref[...], (1, RB))
    ids = lax.broadcasted_iota(jnp.int32, (G, RB), 0)
    onehot = (ids == b).astype(jnp.float32)
    acc_ref[...] += jnp.dot(onehot, h_ref[...], preferred_element_type=jnp.float32)

    @pl.when(i == NBLK - 1)
    def _fin():
        o_ref[...] = (
            jnp.dot(acc_ref[...], fcw_ref[...], preferred_element_type=jnp.float32)
            + fcb_ref[...]
        )


def _pool(h2, batchw, fc_W, fcbr):
    return pl.pallas_call(
        _pool_body,
        grid=(NBLK,),
        in_specs=[
            pl.BlockSpec((RB, D), lambda i: (i, 0)),
            pl.BlockSpec((1, 1, RB), lambda i: (i, 0, 0)),
            pl.BlockSpec((H, H), lambda i: (0, 0)),
            pl.BlockSpec((1, H), lambda i: (0, 0)),
        ],
        out_specs=pl.BlockSpec((G, H), lambda i: (0, 0)),
        out_shape=jax.ShapeDtypeStruct((G, H), jnp.float32),
        scratch_shapes=[pltpu.VMEM((G, H), jnp.float32)],
    )(h2, batchw, fc_W, fcbr)


def kernel(x, edge_index, edge_attr, batch, edge_emb1, edge_emb2, W1, b1, W2, b2, fc_W, fc_b):
    f32 = jnp.float32
    src = edge_index[0]
    dst = edge_index[1]
    code = edge_attr[:, 0] * 3 + edge_attr[:, 1]

    pad = EPAD - E
    srcw = jnp.concatenate([src, jnp.full((pad,), N, jnp.int32)]).reshape(NW, CPT, K)
    dstw = jnp.concatenate([dst, jnp.zeros((pad,), jnp.int32)]).reshape(NW, CPT, K)
    codew = jnp.concatenate([code, jnp.full((pad,), 15, jnp.int32)]).reshape(NW, CPT, K)

    idx = jnp.arange(16)
    T = jnp.where(
        (idx < 9)[:, None],
        edge_emb1[jnp.minimum(idx // 3, 5)] + edge_emb2[idx % 3],
        0.0,
    ).astype(f32)
    TR = jnp.tile(T, (64, 1))
    codew = codew + 16 * (
        jnp.arange(EPAD, dtype=jnp.int32).reshape(NW, CPT, K) & 63
    )

    zrows = jnp.zeros((RPT, D), f32)
    tzero = jnp.zeros((8, D), f32)

    b1p = (b1 + (edge_emb1[0] + edge_emb2[0]) @ W1).reshape(1, 2 * D)
    b2r = b2.reshape(1, H)
    fcbr = fc_b.reshape(1, H)

    P2 = _scatter_pass(TR, codew, dstw, zrows)
    A2 = _scatter_pass(jnp.concatenate([x, tzero], axis=0), srcw, dstw, zrows)
    h1 = _mlp(A2, P2, x, W1, b1p, W2, b2r)
    B2 = _scatter_pass(jnp.concatenate([h1, tzero], axis=0), srcw, dstw, zrows)
    h2 = _mlp(B2, P2, h1, W1, b1p, W2, b2r)

    batchw = batch.reshape(NBLK, 1, RB)
    return _pool(h2, batchw, fc_W, fcbr)
